# baseline (speedup 1.0000x reference)
import functools

import jax
import jax.numpy as jnp
from jax import lax
from jax.experimental import pallas as pl
from jax.experimental.pallas import tpu as pltpu
from jax.experimental.pallas import tpu_sc as plsc

BATCH = 4096
HIST = 50
EMBED = 128

NC = 2
NS = 16
NW = NC * NS

N_ROWS = BATCH * HIST
CHUNK = BATCH // NW
NBUF = 7
PRIME = 4


def _body(xt_hbm, table_hbm, out_hbm, idx_v, rows_v, gsem, wsem):
    wid = lax.axis_index("s") * NC + lax.axis_index("c")
    bbase = wid * CHUNK
    pltpu.sync_copy(xt_hbm.at[:, pl.ds(bbase, CHUNK)], idx_v)

    def start_gather(t, buf):
        pltpu.async_copy(table_hbm.at[idx_v.at[t]], rows_v.at[buf], gsem)

    def wait_gather(t, buf):
        pltpu.make_async_copy(table_hbm.at[idx_v.at[t]], rows_v.at[buf], gsem).wait()

    def start_write(t, buf):
        pltpu.async_copy(
            rows_v.at[buf], out_hbm.at[pl.ds(t * BATCH + bbase, CHUNK)], wsem)

    def wait_write():
        pltpu.make_async_copy(
            rows_v.at[0], out_hbm.at[pl.ds(bbase, CHUNK)], wsem).wait()

    for t in range(PRIME):
        start_gather(t, t)

    def step(t, carry):
        @pl.when(t >= NBUF - PRIME)
        def _():
            wait_write()

        @pl.when(t + PRIME < HIST)
        def _():
            start_gather(t + PRIME, (t + PRIME) % NBUF)

        wait_gather(t, t % NBUF)
        start_write(t, t % NBUF)
        return carry

    lax.fori_loop(0, HIST, step, 0)
    for _ in range(NBUF - PRIME):
        wait_write()


@jax.jit
def _embed(xt, table):
    mesh = plsc.VectorSubcoreMesh(core_axis_name="c", subcore_axis_name="s")
    run = functools.partial(
        pl.kernel,
        out_type=jax.ShapeDtypeStruct((N_ROWS, EMBED), jnp.float32),
        mesh=mesh,
        scratch_types=[
            pltpu.VMEM((HIST, CHUNK), jnp.int32),
            pltpu.VMEM((NBUF, CHUNK, EMBED), jnp.float32),
            pltpu.SemaphoreType.DMA,
            pltpu.SemaphoreType.DMA,
        ],
        compiler_params=pltpu.CompilerParams(use_tc_tiling_on_sc=True),
    )(_body)
    return run(xt, table)


def kernel(x, embedding):
    xt = jnp.swapaxes(jnp.asarray(x, jnp.int32), 0, 1)
    out = _embed(xt, embedding)
    return out.reshape(HIST, BATCH, EMBED).swapaxes(0, 1)


# speedup vs baseline: 1.0046x; 1.0046x over previous
import functools

import jax
import jax.numpy as jnp
from jax import lax
from jax.experimental import pallas as pl
from jax.experimental.pallas import tpu as pltpu
from jax.experimental.pallas import tpu_sc as plsc

BATCH = 4096
HIST = 50
EMBED = 128

NC = 2
NS = 16
NW = NC * NS

N_ROWS = BATCH * HIST
CHUNK = BATCH // NW
NBUF = 6
PRIME = 3


def _body(xt_hbm, table_hbm, out_hbm, idx_v, rows_v, gsem, wsem):
    wid = lax.axis_index("s") * NC + lax.axis_index("c")
    bbase = wid * CHUNK
    pltpu.sync_copy(xt_hbm.at[:, pl.ds(bbase, CHUNK)], idx_v)

    def start_gather(t, buf):
        pltpu.async_copy(table_hbm.at[idx_v.at[t]], rows_v.at[buf], gsem)

    def wait_gather(t, buf):
        pltpu.make_async_copy(table_hbm.at[idx_v.at[t]], rows_v.at[buf], gsem).wait()

    def start_write(t, buf):
        pltpu.async_copy(
            rows_v.at[buf], out_hbm.at[pl.ds(t * BATCH + bbase, CHUNK)], wsem)

    def wait_write():
        pltpu.make_async_copy(
            rows_v.at[0], out_hbm.at[pl.ds(bbase, CHUNK)], wsem).wait()

    for t in range(PRIME):
        start_gather(t, t)

    def step(t, carry):
        @pl.when(t >= NBUF - PRIME)
        def _():
            wait_write()

        @pl.when(t + PRIME < HIST)
        def _():
            start_gather(t + PRIME, (t + PRIME) % NBUF)

        wait_gather(t, t % NBUF)
        start_write(t, t % NBUF)
        return carry

    lax.fori_loop(0, HIST, step, 0)
    for _ in range(NBUF - PRIME):
        wait_write()


@jax.jit
def _embed(xt, table):
    mesh = plsc.VectorSubcoreMesh(core_axis_name="c", subcore_axis_name="s")
    run = functools.partial(
        pl.kernel,
        out_type=jax.ShapeDtypeStruct((N_ROWS, EMBED), jnp.float32),
        mesh=mesh,
        scratch_types=[
            pltpu.VMEM((HIST, CHUNK), jnp.int32),
            pltpu.VMEM((NBUF, CHUNK, EMBED), jnp.float32),
            pltpu.SemaphoreType.DMA,
            pltpu.SemaphoreType.DMA,
        ],
        compiler_params=pltpu.CompilerParams(use_tc_tiling_on_sc=True),
    )(_body)
    return run(xt, table)


def kernel(x, embedding):
    xt = jnp.swapaxes(jnp.asarray(x, jnp.int32), 0, 1)
    out = _embed(xt, embedding)
    return out.reshape(HIST, BATCH, EMBED).swapaxes(0, 1)
